# Initial kernel scaffold; baseline (speedup 1.0000x reference)
import jax
import jax.numpy as jnp
from jax.experimental import pallas as pl


def kernel(efeat, nfeat, edge_index, eW1, eb1, eW2, eb2, eln_g, eln_b, nW1, nb1, nW2, nb2, nln_g, nln_b):
    raise NotImplementedError("write your pallas kernel here")


# speedup vs baseline: 2.8890x; 2.8890x over previous
import functools

import jax
import jax.numpy as jnp
from jax import lax
from jax.experimental import pallas as pl
from jax.experimental.pallas import tpu as pltpu
from jax.experimental.pallas import tpu_sc as plsc

_N = 10000
_E = 320000
_D = 128
_L = 4

_NC = 2
_NS = 16
_NW = _NC * _NS
_PER_W = _E // _NW
_B = 80
_NB = _PER_W // _B
_NPAD = 10240
_ZROWS = 128
_AGG_T = _NPAD // _NS
_BLK_E = 4000
_BLK_N = 1000


def _sc_gather(ps, pd, src, dst):
    mesh = plsc.VectorSubcoreMesh(core_axis_name="c", subcore_axis_name="s")

    @functools.partial(
        pl.kernel,
        out_type=jax.ShapeDtypeStruct((_E, _D), jnp.float32),
        mesh=mesh,
        scratch_types=[
            pltpu.VMEM((_B,), jnp.int32),
            pltpu.VMEM((_B,), jnp.int32),
            pltpu.VMEM((_B, _D), jnp.float32),
            pltpu.VMEM((_B, _D), jnp.float32),
            pltpu.SemaphoreType.DMA,
            pltpu.SemaphoreType.DMA,
        ],
    )
    def k(ps_h, pd_h, src_h, dst_h, g_h, idx_s, idx_d, rows_s, rows_d, sem_s, sem_d):
        c = lax.axis_index("c")
        s = lax.axis_index("s")
        base0 = (c * _NS + s) * _PER_W

        def batch(b, carry):
            base = base0 + b * _B
            pltpu.sync_copy(src_h.at[pl.ds(base, _B)], idx_s)
            pltpu.sync_copy(dst_h.at[pl.ds(base, _B)], idx_d)
            cp_s = pltpu.async_copy(ps_h.at[idx_s], rows_s, sem_s)
            cp_d = pltpu.async_copy(pd_h.at[idx_d], rows_d, sem_d)
            cp_s.wait()
            cp_d.wait()

            def add_row(j, carry2):
                for t in range(_D // 16):
                    sl = pl.ds(t * 16, 16)
                    rows_s[j, sl] = rows_s[j, sl] + rows_d[j, sl]
                return carry2

            lax.fori_loop(0, _B, add_row, 0)
            pltpu.sync_copy(rows_s, g_h.at[pl.ds(base, _B)])
            return carry

        lax.fori_loop(0, _NB, batch, 0)

    return k(ps, pd, src, dst)


def _sc_scatter(efeat, dst):
    mesh = plsc.VectorSubcoreMesh(core_axis_name="c", subcore_axis_name="s")

    @functools.partial(
        pl.kernel,
        out_type=jax.ShapeDtypeStruct((_NC, _NPAD, _D), jnp.float32),
        mesh=mesh,
        scratch_types=[
            pltpu.VMEM((_B,), jnp.int32),
            pltpu.VMEM((_B, _D), jnp.float32),
            pltpu.VMEM((_ZROWS, _D), jnp.float32),
            pltpu.VMEM_SHARED((_NPAD, _D), jnp.float32),
        ],
    )
    def k(ef_h, dst_h, agg_h, idx_d, rows, stage, agg_sh):
        c = lax.axis_index("c")
        s = lax.axis_index("s")
        base0 = (c * _NS + s) * _PER_W

        def zrow(j, carry):
            for t in range(_D // 16):
                stage[j, pl.ds(t * 16, 16)] = jnp.zeros((16,), jnp.float32)
            return carry

        lax.fori_loop(0, _ZROWS, zrow, 0)
        for ch in range(_AGG_T // _ZROWS):
            pltpu.sync_copy(stage, agg_sh.at[pl.ds(s * _AGG_T + ch * _ZROWS, _ZROWS)])
        plsc.subcore_barrier()

        def batch(b, carry):
            base = base0 + b * _B
            pltpu.sync_copy(dst_h.at[pl.ds(base, _B)], idx_d)
            pltpu.sync_copy(ef_h.at[pl.ds(base, _B)], rows)
            pltpu.sync_copy(rows, agg_sh.at[idx_d], add=True)
            return carry

        lax.fori_loop(0, _NB, batch, 0)
        plsc.subcore_barrier()

        for ch in range(_AGG_T // _ZROWS):
            off = s * _AGG_T + ch * _ZROWS
            pltpu.sync_copy(agg_sh.at[pl.ds(off, _ZROWS)], stage)
            pltpu.sync_copy(stage, agg_h.at[c, pl.ds(off, _ZROWS)])

    return k(efeat, dst)


def _tc_proj(nfeat, ws, wd):
    blk = _BLK_N

    def body(nf_r, ws_r, wd_r, ps_r, pd_r):
        nf = nf_r[...]
        ps_r[...] = jnp.dot(nf, ws_r[...], preferred_element_type=jnp.float32)
        pd_r[...] = jnp.dot(nf, wd_r[...], preferred_element_type=jnp.float32)

    return pl.pallas_call(
        body,
        grid=(_N // blk,),
        in_specs=[
            pl.BlockSpec((blk, _D), lambda i: (i, 0)),
            pl.BlockSpec((_D, _D), lambda i: (0, 0)),
            pl.BlockSpec((_D, _D), lambda i: (0, 0)),
        ],
        out_specs=[
            pl.BlockSpec((blk, _D), lambda i: (i, 0)),
            pl.BlockSpec((blk, _D), lambda i: (i, 0)),
        ],
        out_shape=[
            jax.ShapeDtypeStruct((_N, _D), jnp.float32),
            jax.ShapeDtypeStruct((_N, _D), jnp.float32),
        ],
    )(nfeat, ws, wd)


def _ln_res(base, h, g, b):
    mu = jnp.mean(h, axis=-1, keepdims=True)
    var = jnp.mean((h - mu) ** 2, axis=-1, keepdims=True)
    return base + ((h - mu) * lax.rsqrt(var + 1e-5)) * g + b


def _tc_edge(efeat, g, we, w2, b1, b2, lng, lnb):
    blk = _BLK_E

    def body(ef_r, g_r, we_r, w2_r, b1_r, b2_r, lng_r, lnb_r, out_r):
        ef = ef_r[...]
        x = jnp.dot(ef, we_r[...], preferred_element_type=jnp.float32)
        x = x + g_r[...] + b1_r[...]
        x = x * jax.nn.sigmoid(x)
        h = jnp.dot(x, w2_r[...], preferred_element_type=jnp.float32) + b2_r[...]
        out_r[...] = _ln_res(ef, h, lng_r[...], lnb_r[...])

    vec = lambda a: a.reshape(1, _D)
    return pl.pallas_call(
        body,
        grid=(_E // blk,),
        in_specs=[
            pl.BlockSpec((blk, _D), lambda i: (i, 0)),
            pl.BlockSpec((blk, _D), lambda i: (i, 0)),
            pl.BlockSpec((_D, _D), lambda i: (0, 0)),
            pl.BlockSpec((_D, _D), lambda i: (0, 0)),
            pl.BlockSpec((1, _D), lambda i: (0, 0)),
            pl.BlockSpec((1, _D), lambda i: (0, 0)),
            pl.BlockSpec((1, _D), lambda i: (0, 0)),
            pl.BlockSpec((1, _D), lambda i: (0, 0)),
        ],
        out_specs=pl.BlockSpec((blk, _D), lambda i: (i, 0)),
        out_shape=jax.ShapeDtypeStruct((_E, _D), jnp.float32),
    )(efeat, g, we, w2, vec(b1), vec(b2), vec(lng), vec(lnb))


def _tc_node(nfeat, aggs, w1a, w1b, w2, b1, b2, lng, lnb, ws_next, wd_next):
    blk = _BLK_N

    def body(nf_r, a0_r, a1_r, w1a_r, w1b_r, w2_r, b1_r, b2_r, lng_r, lnb_r,
             wsn_r, wdn_r, nf_o, ps_o, pd_o):
        nf = nf_r[...]
        agg = a0_r[0] + a1_r[0]
        x = jnp.dot(nf, w1a_r[...], preferred_element_type=jnp.float32)
        x = x + jnp.dot(agg, w1b_r[...], preferred_element_type=jnp.float32)
        x = x + b1_r[...]
        x = x * jax.nn.sigmoid(x)
        h = jnp.dot(x, w2_r[...], preferred_element_type=jnp.float32) + b2_r[...]
        nf_new = _ln_res(nf, h, lng_r[...], lnb_r[...])
        nf_o[...] = nf_new
        ps_o[...] = jnp.dot(nf_new, wsn_r[...], preferred_element_type=jnp.float32)
        pd_o[...] = jnp.dot(nf_new, wdn_r[...], preferred_element_type=jnp.float32)

    vec = lambda a: a.reshape(1, _D)
    row = pl.BlockSpec((blk, _D), lambda i: (i, 0))
    wsp = pl.BlockSpec((_D, _D), lambda i: (0, 0))
    bsp = pl.BlockSpec((1, _D), lambda i: (0, 0))
    return pl.pallas_call(
        body,
        grid=(_N // blk,),
        in_specs=[
            row,
            pl.BlockSpec((1, blk, _D), lambda i: (0, i, 0)),
            pl.BlockSpec((1, blk, _D), lambda i: (1, i, 0)),
            wsp, wsp, wsp, bsp, bsp, bsp, bsp, wsp, wsp,
        ],
        out_specs=[row, row, row],
        out_shape=[
            jax.ShapeDtypeStruct((_N, _D), jnp.float32),
            jax.ShapeDtypeStruct((_N, _D), jnp.float32),
            jax.ShapeDtypeStruct((_N, _D), jnp.float32),
        ],
    )(nfeat, aggs, aggs, w1a, w1b, w2, vec(b1), vec(b2), vec(lng), vec(lnb),
      ws_next, wd_next)


def kernel(efeat, nfeat, edge_index, eW1, eb1, eW2, eb2, eln_g, eln_b,
           nW1, nb1, nW2, nb2, nln_g, nln_b):
    src = edge_index[0]
    dst = edge_index[1]
    ps, pd = _tc_proj(nfeat, eW1[0, _D:2 * _D], eW1[0, 2 * _D:])
    for i in range(_L):
        g = _sc_gather(ps, pd, src, dst)
        efeat = _tc_edge(efeat, g, eW1[i, :_D], eW2[i], eb1[i], eb2[i],
                         eln_g[i], eln_b[i])
        aggs = _sc_scatter(efeat, dst)
        j = (i + 1) % _L
        nfeat, ps, pd = _tc_node(nfeat, aggs, nW1[i, :_D], nW1[i, _D:], nW2[i],
                                 nb1[i], nb2[i], nln_g[i], nln_b[i],
                                 eW1[j, _D:2 * _D], eW1[j, 2 * _D:])
    return (efeat, nfeat)


# speedup vs baseline: 5.1150x; 1.7705x over previous
import functools

import jax
import jax.numpy as jnp
from jax import lax
from jax.experimental import pallas as pl
from jax.experimental.pallas import tpu as pltpu
from jax.experimental.pallas import tpu_sc as plsc

_N = 10000
_E = 320000
_D = 128
_L = 4

_NC = 2
_NS = 16
_NW = _NC * _NS
_PER_W = _E // _NW
_B = 80
_NB = _PER_W // _B
_NPAD = 10240
_ZROWS = 128
_AGG_T = _NPAD // _NS
_BLK_E = 4000
_BLK_N = 1000


def _sc_gather(ps, pd, src, dst):
    mesh = plsc.VectorSubcoreMesh(core_axis_name="c", subcore_axis_name="s")
    S = 3

    @functools.partial(
        pl.kernel,
        out_type=jax.ShapeDtypeStruct((_E, _D), jnp.float32),
        mesh=mesh,
        scratch_types=(
            [pltpu.VMEM((_PER_W,), jnp.int32)] * 2
            + [pltpu.VMEM((_B, _D), jnp.float32)] * (3 * S)
            + [pltpu.SemaphoreType.DMA] * (2 * S)
        ),
    )
    def k(ps_h, pd_h, src_h, dst_h, g_h, idx_s, idx_d, *bufs):
        rows_s = bufs[0:S]
        rows_d = bufs[S:2 * S]
        out = bufs[2 * S:3 * S]
        sem_g = bufs[3 * S:4 * S]
        sem_st = bufs[4 * S:5 * S]
        c = lax.axis_index("c")
        s = lax.axis_index("s")
        base0 = (c * _NS + s) * _PER_W
        pltpu.sync_copy(src_h.at[pl.ds(base0, _PER_W)], idx_s)
        pltpu.sync_copy(dst_h.at[pl.ds(base0, _PER_W)], idx_d)

        def issue(b, j):
            isl = pl.ds(b * _B, _B)
            pltpu.async_copy(ps_h.at[idx_s.at[isl]], rows_s[j], sem_g[j])
            pltpu.async_copy(pd_h.at[idx_d.at[isl]], rows_d[j], sem_g[j])

        def drain_gather(j):
            pltpu.make_async_copy(ps_h.at[pl.ds(0, _B)], rows_s[j], sem_g[j]).wait()
            pltpu.make_async_copy(pd_h.at[pl.ds(0, _B)], rows_d[j], sem_g[j]).wait()

        def drain_store(j):
            pltpu.make_async_copy(out[j], g_h.at[pl.ds(0, _B)], sem_st[j]).wait()

        def step(b, j, wait_st, look):
            drain_gather(j)
            if wait_st:
                drain_store(j)

            def add_row(r, carry):
                for t in range(_D // 16):
                    sl = pl.ds(t * 16, 16)
                    out[j][r, sl] = rows_s[j][r, sl] + rows_d[j][r, sl]
                return carry

            lax.fori_loop(0, _B, add_row, 0)
            pltpu.async_copy(out[j], g_h.at[pl.ds(base0 + b * _B, _B)], sem_st[j])
            if look:
                issue(b + S - 1, (j + S - 1) % S)

        for j in range(S - 1):
            issue(j, j)
        for j in range(S):
            step(j, j, wait_st=False, look=True)

        def body(ii, carry):
            b = ii * S
            for j in range(S):
                step(b + j, j, wait_st=True, look=True)
            return carry

        n_mid = _NB // S - 2
        lax.fori_loop(1, 1 + n_mid, body, 0)
        tail0 = (_NB // S - 1) * S
        for b in range(tail0, _NB):
            step(b, b % S, wait_st=True, look=(b + S - 1) < _NB)
        for j in range(S):
            drain_store(j)

    return k(ps, pd, src, dst)


def _sc_scatter(efeat, dst):
    mesh = plsc.VectorSubcoreMesh(core_axis_name="c", subcore_axis_name="s")
    S = 4

    @functools.partial(
        pl.kernel,
        out_type=jax.ShapeDtypeStruct((_NC, _NPAD, _D), jnp.float32),
        mesh=mesh,
        scratch_types=(
            [pltpu.VMEM((_B,), jnp.int32)] * S
            + [pltpu.VMEM((_B, _D), jnp.float32)] * S
            + [pltpu.VMEM_SHARED((_NPAD, _D), jnp.float32)]
            + [pltpu.SemaphoreType.DMA] * (2 * S)
        ),
    )
    def k(ef_h, dst_h, agg_h, *bufs):
        idx = bufs[0:S]
        rows = bufs[S:2 * S]
        agg_sh = bufs[2 * S]
        sem_l = bufs[2 * S + 1:3 * S + 1]
        sem_sc = bufs[3 * S + 1:4 * S + 1]
        c = lax.axis_index("c")
        s = lax.axis_index("s")
        base0 = (c * _NS + s) * _PER_W

        def zrow(r, carry):
            for t in range(_D // 16):
                rows[0][r, pl.ds(t * 16, 16)] = jnp.zeros((16,), jnp.float32)
            return carry

        lax.fori_loop(0, _B, zrow, 0)
        for ch in range(_AGG_T // _B):
            pltpu.sync_copy(rows[0], agg_sh.at[pl.ds(s * _AGG_T + ch * _B, _B)])
        plsc.subcore_barrier()

        def load(b, j):
            base = pl.ds(base0 + b * _B, _B)
            pltpu.async_copy(dst_h.at[base], idx[j], sem_l[j])
            pltpu.async_copy(ef_h.at[base], rows[j], sem_l[j])

        def drain_load(j):
            pltpu.make_async_copy(dst_h.at[pl.ds(0, _B)], idx[j], sem_l[j]).wait()
            pltpu.make_async_copy(ef_h.at[pl.ds(0, _B)], rows[j], sem_l[j]).wait()

        def drain_scatter(j):
            pltpu.make_async_copy(ef_h.at[pl.ds(0, _B)], rows[j], sem_sc[j]).wait()

        def step(b, j, wait_sc, look):
            drain_load(j)
            pltpu.async_copy(rows[j], agg_sh.at[idx[j]], sem_sc[j], add=True)
            if wait_sc:
                drain_scatter((j + S - 2) % S)
            if look:
                load(b + S - 2, (j + S - 2) % S)

        for j in range(S - 2):
            load(j, j)
        for j in range(S):
            step(j, j, wait_sc=j >= 2, look=True)

        def body(ii, carry):
            b = ii * S
            for j in range(S):
                step(b + j, j, wait_sc=True, look=True)
            return carry

        lax.fori_loop(1, _NB // S - 1, body, 0)
        for b in range((_NB // S - 1) * S, _NB):
            step(b, b % S, wait_sc=True, look=(b + S - 2) < _NB)
        drain_scatter((_NB - 1) % S)
        drain_scatter((_NB - 2) % S)
        plsc.subcore_barrier()

        for ch in range(_AGG_T // _B):
            off = s * _AGG_T + ch * _B
            pltpu.sync_copy(agg_sh.at[pl.ds(off, _B)], rows[0])
            pltpu.sync_copy(rows[0], agg_h.at[c, pl.ds(off, _B)])

    return k(efeat, dst)


def _tc_proj(nfeat, ws, wd):
    blk = _BLK_N

    def body(nf_r, ws_r, wd_r, ps_r, pd_r):
        nf = nf_r[...]
        ps_r[...] = jnp.dot(nf, ws_r[...], preferred_element_type=jnp.float32)
        pd_r[...] = jnp.dot(nf, wd_r[...], preferred_element_type=jnp.float32)

    return pl.pallas_call(
        body,
        grid=(_N // blk,),
        in_specs=[
            pl.BlockSpec((blk, _D), lambda i: (i, 0)),
            pl.BlockSpec((_D, _D), lambda i: (0, 0)),
            pl.BlockSpec((_D, _D), lambda i: (0, 0)),
        ],
        out_specs=[
            pl.BlockSpec((blk, _D), lambda i: (i, 0)),
            pl.BlockSpec((blk, _D), lambda i: (i, 0)),
        ],
        out_shape=[
            jax.ShapeDtypeStruct((_N, _D), jnp.float32),
            jax.ShapeDtypeStruct((_N, _D), jnp.float32),
        ],
    )(nfeat, ws, wd)


def _ln_res(base, h, g, b):
    mu = jnp.mean(h, axis=-1, keepdims=True)
    var = jnp.mean((h - mu) ** 2, axis=-1, keepdims=True)
    return base + ((h - mu) * lax.rsqrt(var + 1e-5)) * g + b


def _tc_edge(efeat, g, we, w2, b1, b2, lng, lnb):
    blk = _BLK_E

    def body(ef_r, g_r, we_r, w2_r, b1_r, b2_r, lng_r, lnb_r, out_r):
        ef = ef_r[...]
        x = jnp.dot(ef, we_r[...], preferred_element_type=jnp.float32)
        x = x + g_r[...] + b1_r[...]
        x = x * jax.nn.sigmoid(x)
        h = jnp.dot(x, w2_r[...], preferred_element_type=jnp.float32) + b2_r[...]
        out_r[...] = _ln_res(ef, h, lng_r[...], lnb_r[...])

    vec = lambda a: a.reshape(1, _D)
    return pl.pallas_call(
        body,
        grid=(_E // blk,),
        in_specs=[
            pl.BlockSpec((blk, _D), lambda i: (i, 0)),
            pl.BlockSpec((blk, _D), lambda i: (i, 0)),
            pl.BlockSpec((_D, _D), lambda i: (0, 0)),
            pl.BlockSpec((_D, _D), lambda i: (0, 0)),
            pl.BlockSpec((1, _D), lambda i: (0, 0)),
            pl.BlockSpec((1, _D), lambda i: (0, 0)),
            pl.BlockSpec((1, _D), lambda i: (0, 0)),
            pl.BlockSpec((1, _D), lambda i: (0, 0)),
        ],
        out_specs=pl.BlockSpec((blk, _D), lambda i: (i, 0)),
        out_shape=jax.ShapeDtypeStruct((_E, _D), jnp.float32),
    )(efeat, g, we, w2, vec(b1), vec(b2), vec(lng), vec(lnb))


def _tc_node(nfeat, aggs, w1a, w1b, w2, b1, b2, lng, lnb, ws_next, wd_next):
    blk = _BLK_N

    def body(nf_r, a0_r, a1_r, w1a_r, w1b_r, w2_r, b1_r, b2_r, lng_r, lnb_r,
             wsn_r, wdn_r, nf_o, ps_o, pd_o):
        nf = nf_r[...]
        agg = a0_r[0] + a1_r[0]
        x = jnp.dot(nf, w1a_r[...], preferred_element_type=jnp.float32)
        x = x + jnp.dot(agg, w1b_r[...], preferred_element_type=jnp.float32)
        x = x + b1_r[...]
        x = x * jax.nn.sigmoid(x)
        h = jnp.dot(x, w2_r[...], preferred_element_type=jnp.float32) + b2_r[...]
        nf_new = _ln_res(nf, h, lng_r[...], lnb_r[...])
        nf_o[...] = nf_new
        ps_o[...] = jnp.dot(nf_new, wsn_r[...], preferred_element_type=jnp.float32)
        pd_o[...] = jnp.dot(nf_new, wdn_r[...], preferred_element_type=jnp.float32)

    vec = lambda a: a.reshape(1, _D)
    row = pl.BlockSpec((blk, _D), lambda i: (i, 0))
    wsp = pl.BlockSpec((_D, _D), lambda i: (0, 0))
    bsp = pl.BlockSpec((1, _D), lambda i: (0, 0))
    return pl.pallas_call(
        body,
        grid=(_N // blk,),
        in_specs=[
            row,
            pl.BlockSpec((1, blk, _D), lambda i: (0, i, 0)),
            pl.BlockSpec((1, blk, _D), lambda i: (1, i, 0)),
            wsp, wsp, wsp, bsp, bsp, bsp, bsp, wsp, wsp,
        ],
        out_specs=[row, row, row],
        out_shape=[
            jax.ShapeDtypeStruct((_N, _D), jnp.float32),
            jax.ShapeDtypeStruct((_N, _D), jnp.float32),
            jax.ShapeDtypeStruct((_N, _D), jnp.float32),
        ],
    )(nfeat, aggs, aggs, w1a, w1b, w2, vec(b1), vec(b2), vec(lng), vec(lnb),
      ws_next, wd_next)


def kernel(efeat, nfeat, edge_index, eW1, eb1, eW2, eb2, eln_g, eln_b,
           nW1, nb1, nW2, nb2, nln_g, nln_b):
    src = edge_index[0]
    dst = edge_index[1]
    ps, pd = _tc_proj(nfeat, eW1[0, _D:2 * _D], eW1[0, 2 * _D:])
    for i in range(_L):
        g = _sc_gather(ps, pd, src, dst)
        efeat = _tc_edge(efeat, g, eW1[i, :_D], eW2[i], eb1[i], eb2[i],
                         eln_g[i], eln_b[i])
        aggs = _sc_scatter(efeat, dst)
        j = (i + 1) % _L
        nfeat, ps, pd = _tc_node(nfeat, aggs, nW1[i, :_D], nW1[i, _D:], nW2[i],
                                 nb1[i], nb2[i], nln_g[i], nln_b[i],
                                 eW1[j, _D:2 * _D], eW1[j, 2 * _D:])
    return (efeat, nfeat)


# speedup vs baseline: 5.1677x; 1.0103x over previous
import functools

import jax
import jax.numpy as jnp
from jax import lax
from jax.experimental import pallas as pl
from jax.experimental.pallas import tpu as pltpu
from jax.experimental.pallas import tpu_sc as plsc

_N = 10000
_E = 320000
_D = 128
_L = 4

_NC = 2
_NS = 16
_NW = _NC * _NS
_PER_W = _E // _NW
_B = 80
_NB = _PER_W // _B
_NPAD = 10240
_ZROWS = 128
_AGG_T = _NPAD // _NS
_BLK_N = 1000
_EDGE_GRID = 40


def _sc_gather(ps, pd, src, dst):
    cnt = src.shape[0]
    per_w = cnt // _NW
    nb = per_w // _B
    mesh = plsc.VectorSubcoreMesh(core_axis_name="c", subcore_axis_name="s")
    S = 3

    @functools.partial(
        pl.kernel,
        out_type=jax.ShapeDtypeStruct((cnt, _D), jnp.float32),
        mesh=mesh,
        scratch_types=(
            [pltpu.VMEM((per_w,), jnp.int32)] * 2
            + [pltpu.VMEM((_B, _D), jnp.float32)] * (3 * S)
            + [pltpu.SemaphoreType.DMA] * (2 * S)
        ),
    )
    def k(ps_h, pd_h, src_h, dst_h, g_h, idx_s, idx_d, *bufs):
        rows_s = bufs[0:S]
        rows_d = bufs[S:2 * S]
        out = bufs[2 * S:3 * S]
        sem_g = bufs[3 * S:4 * S]
        sem_st = bufs[4 * S:5 * S]
        c = lax.axis_index("c")
        s = lax.axis_index("s")
        base0 = (c * _NS + s) * per_w
        pltpu.sync_copy(src_h.at[pl.ds(base0, per_w)], idx_s)
        pltpu.sync_copy(dst_h.at[pl.ds(base0, per_w)], idx_d)

        def issue(b, j):
            isl = pl.ds(b * _B, _B)
            pltpu.async_copy(ps_h.at[idx_s.at[isl]], rows_s[j], sem_g[j])
            pltpu.async_copy(pd_h.at[idx_d.at[isl]], rows_d[j], sem_g[j])

        def drain_gather(j):
            pltpu.make_async_copy(ps_h.at[pl.ds(0, _B)], rows_s[j], sem_g[j]).wait()
            pltpu.make_async_copy(pd_h.at[pl.ds(0, _B)], rows_d[j], sem_g[j]).wait()

        def drain_store(j):
            pltpu.make_async_copy(out[j], g_h.at[pl.ds(0, _B)], sem_st[j]).wait()

        def step(b, j, wait_st, look):
            drain_gather(j)
            if wait_st:
                drain_store(j)

            def add_row(r, carry):
                for t in range(_D // 16):
                    sl = pl.ds(t * 16, 16)
                    out[j][r, sl] = rows_s[j][r, sl] + rows_d[j][r, sl]
                return carry

            lax.fori_loop(0, _B, add_row, 0)
            pltpu.async_copy(out[j], g_h.at[pl.ds(base0 + b * _B, _B)], sem_st[j])
            if look:
                issue(b + S - 1, (j + S - 1) % S)

        for j in range(S - 1):
            issue(j, j)
        for j in range(S):
            step(j, j, wait_st=False, look=True)

        def body(ii, carry):
            b = ii * S
            for j in range(S):
                step(b + j, j, wait_st=True, look=True)
            return carry

        n_mid = nb // S - 2
        lax.fori_loop(1, 1 + n_mid, body, 0)
        tail0 = (nb // S - 1) * S
        for b in range(tail0, nb):
            step(b, b % S, wait_st=True, look=(b + S - 1) < nb)
        for j in range(S):
            drain_store(j)

    return k(ps, pd, src, dst)


def _sc_scatter(efeat, dst):
    cnt = dst.shape[0]
    per_w = cnt // _NW
    nb = per_w // _B
    mesh = plsc.VectorSubcoreMesh(core_axis_name="c", subcore_axis_name="s")
    S = 4

    @functools.partial(
        pl.kernel,
        out_type=jax.ShapeDtypeStruct((_NC, _NPAD, _D), jnp.float32),
        mesh=mesh,
        scratch_types=(
            [pltpu.VMEM((_B,), jnp.int32)] * S
            + [pltpu.VMEM((_B, _D), jnp.float32)] * S
            + [pltpu.VMEM_SHARED((_NPAD, _D), jnp.float32)]
            + [pltpu.SemaphoreType.DMA] * (2 * S)
        ),
    )
    def k(ef_h, dst_h, agg_h, *bufs):
        idx = bufs[0:S]
        rows = bufs[S:2 * S]
        agg_sh = bufs[2 * S]
        sem_l = bufs[2 * S + 1:3 * S + 1]
        sem_sc = bufs[3 * S + 1:4 * S + 1]
        c = lax.axis_index("c")
        s = lax.axis_index("s")
        base0 = (c * _NS + s) * per_w

        def zrow(r, carry):
            for t in range(_D // 16):
                rows[0][r, pl.ds(t * 16, 16)] = jnp.zeros((16,), jnp.float32)
            return carry

        lax.fori_loop(0, _B, zrow, 0)
        for ch in range(_AGG_T // _B):
            pltpu.sync_copy(rows[0], agg_sh.at[pl.ds(s * _AGG_T + ch * _B, _B)])
        plsc.subcore_barrier()

        def load(b, j):
            base = pl.ds(base0 + b * _B, _B)
            pltpu.async_copy(dst_h.at[base], idx[j], sem_l[j])
            pltpu.async_copy(ef_h.at[base], rows[j], sem_l[j])

        def drain_load(j):
            pltpu.make_async_copy(dst_h.at[pl.ds(0, _B)], idx[j], sem_l[j]).wait()
            pltpu.make_async_copy(ef_h.at[pl.ds(0, _B)], rows[j], sem_l[j]).wait()

        def drain_scatter(j):
            pltpu.make_async_copy(ef_h.at[pl.ds(0, _B)], rows[j], sem_sc[j]).wait()

        def step(b, j, wait_sc, look):
            drain_load(j)
            pltpu.async_copy(rows[j], agg_sh.at[idx[j]], sem_sc[j], add=True)
            if wait_sc:
                drain_scatter((j + S - 2) % S)
            if look:
                load(b + S - 2, (j + S - 2) % S)

        for j in range(S - 2):
            load(j, j)
        for j in range(S):
            step(j, j, wait_sc=j >= 2, look=True)

        def body(ii, carry):
            b = ii * S
            for j in range(S):
                step(b + j, j, wait_sc=True, look=True)
            return carry

        lax.fori_loop(1, nb // S - 1, body, 0)
        for b in range((nb // S - 1) * S, nb):
            step(b, b % S, wait_sc=True, look=(b + S - 2) < nb)
        drain_scatter((nb - 1) % S)
        drain_scatter((nb - 2) % S)
        plsc.subcore_barrier()

        for ch in range(_AGG_T // _B):
            off = s * _AGG_T + ch * _B
            pltpu.sync_copy(agg_sh.at[pl.ds(off, _B)], rows[0])
            pltpu.sync_copy(rows[0], agg_h.at[c, pl.ds(off, _B)])

    return k(efeat, dst)


def _tc_proj(nfeat, ws, wd):
    blk = _BLK_N

    def body(nf_r, ws_r, wd_r, ps_r, pd_r):
        nf = nf_r[...]
        ps_r[...] = jnp.dot(nf, ws_r[...], preferred_element_type=jnp.float32)
        pd_r[...] = jnp.dot(nf, wd_r[...], preferred_element_type=jnp.float32)

    return pl.pallas_call(
        body,
        grid=(_N // blk,),
        in_specs=[
            pl.BlockSpec((blk, _D), lambda i: (i, 0)),
            pl.BlockSpec((_D, _D), lambda i: (0, 0)),
            pl.BlockSpec((_D, _D), lambda i: (0, 0)),
        ],
        out_specs=[
            pl.BlockSpec((blk, _D), lambda i: (i, 0)),
            pl.BlockSpec((blk, _D), lambda i: (i, 0)),
        ],
        out_shape=[
            jax.ShapeDtypeStruct((_N, _D), jnp.float32),
            jax.ShapeDtypeStruct((_N, _D), jnp.float32),
        ],
    )(nfeat, ws, wd)


def _ln_res(base, h, g, b):
    mu = jnp.mean(h, axis=-1, keepdims=True)
    var = jnp.mean((h - mu) ** 2, axis=-1, keepdims=True)
    return base + ((h - mu) * lax.rsqrt(var + 1e-5)) * g + b


def _tc_edge(efeat, g, we, w2, b1, b2, lng, lnb):
    cnt = efeat.shape[0]
    blk = cnt // _EDGE_GRID

    def body(ef_r, g_r, we_r, w2_r, b1_r, b2_r, lng_r, lnb_r, out_r):
        ef = ef_r[...]
        x = jnp.dot(ef, we_r[...], preferred_element_type=jnp.float32)
        x = x + g_r[...] + b1_r[...]
        x = x * jax.nn.sigmoid(x)
        h = jnp.dot(x, w2_r[...], preferred_element_type=jnp.float32) + b2_r[...]
        out_r[...] = _ln_res(ef, h, lng_r[...], lnb_r[...])

    vec = lambda a: a.reshape(1, _D)
    return pl.pallas_call(
        body,
        grid=(cnt // blk,),
        in_specs=[
            pl.BlockSpec((blk, _D), lambda i: (i, 0)),
            pl.BlockSpec((blk, _D), lambda i: (i, 0)),
            pl.BlockSpec((_D, _D), lambda i: (0, 0)),
            pl.BlockSpec((_D, _D), lambda i: (0, 0)),
            pl.BlockSpec((1, _D), lambda i: (0, 0)),
            pl.BlockSpec((1, _D), lambda i: (0, 0)),
            pl.BlockSpec((1, _D), lambda i: (0, 0)),
            pl.BlockSpec((1, _D), lambda i: (0, 0)),
        ],
        out_specs=pl.BlockSpec((blk, _D), lambda i: (i, 0)),
        out_shape=jax.ShapeDtypeStruct((cnt, _D), jnp.float32),
    )(efeat, g, we, w2, vec(b1), vec(b2), vec(lng), vec(lnb))


def _tc_node(nfeat, aggs0, aggs1, w1a, w1b, w2, b1, b2, lng, lnb,
             ws_next, wd_next):
    blk = _BLK_N

    def body(nf_r, a0_r, a1_r, a2_r, a3_r, w1a_r, w1b_r, w2_r, b1_r, b2_r,
             lng_r, lnb_r, wsn_r, wdn_r, nf_o, ps_o, pd_o):
        nf = nf_r[...]
        agg = (a0_r[0] + a1_r[0]) + (a2_r[0] + a3_r[0])
        x = jnp.dot(nf, w1a_r[...], preferred_element_type=jnp.float32)
        x = x + jnp.dot(agg, w1b_r[...], preferred_element_type=jnp.float32)
        x = x + b1_r[...]
        x = x * jax.nn.sigmoid(x)
        h = jnp.dot(x, w2_r[...], preferred_element_type=jnp.float32) + b2_r[...]
        nf_new = _ln_res(nf, h, lng_r[...], lnb_r[...])
        nf_o[...] = nf_new
        ps_o[...] = jnp.dot(nf_new, wsn_r[...], preferred_element_type=jnp.float32)
        pd_o[...] = jnp.dot(nf_new, wdn_r[...], preferred_element_type=jnp.float32)

    vec = lambda a: a.reshape(1, _D)
    row = pl.BlockSpec((blk, _D), lambda i: (i, 0))
    wsp = pl.BlockSpec((_D, _D), lambda i: (0, 0))
    bsp = pl.BlockSpec((1, _D), lambda i: (0, 0))
    return pl.pallas_call(
        body,
        grid=(_N // blk,),
        in_specs=[
            row,
            pl.BlockSpec((1, blk, _D), lambda i: (0, i, 0)),
            pl.BlockSpec((1, blk, _D), lambda i: (1, i, 0)),
            pl.BlockSpec((1, blk, _D), lambda i: (0, i, 0)),
            pl.BlockSpec((1, blk, _D), lambda i: (1, i, 0)),
            wsp, wsp, wsp, bsp, bsp, bsp, bsp, wsp, wsp,
        ],
        out_specs=[row, row, row],
        out_shape=[
            jax.ShapeDtypeStruct((_N, _D), jnp.float32),
            jax.ShapeDtypeStruct((_N, _D), jnp.float32),
            jax.ShapeDtypeStruct((_N, _D), jnp.float32),
        ],
    )(nfeat, aggs0, aggs0, aggs1, aggs1, w1a, w1b, w2, vec(b1), vec(b2),
      vec(lng), vec(lnb), ws_next, wd_next)


_C0 = 158720


def kernel(efeat, nfeat, edge_index, eW1, eb1, eW2, eb2, eln_g, eln_b,
           nW1, nb1, nW2, nb2, nln_g, nln_b):
    src = edge_index[0]
    dst = edge_index[1]
    srcs = (src[:_C0], src[_C0:])
    dsts = (dst[:_C0], dst[_C0:])
    efs = [efeat[:_C0], efeat[_C0:]]
    ps, pd = _tc_proj(nfeat, eW1[0, _D:2 * _D], eW1[0, 2 * _D:])
    for i in range(_L):
        gs = [None, None]
        aggs = [None, None]
        for k in range(2):
            gs[k] = _sc_gather(ps, pd, srcs[k], dsts[k])
        for k in range(2):
            efs[k] = _tc_edge(efs[k], gs[k], eW1[i, :_D], eW2[i], eb1[i],
                              eb2[i], eln_g[i], eln_b[i])
            aggs[k] = _sc_scatter(efs[k], dsts[k])
        j = (i + 1) % _L
        nfeat, ps, pd = _tc_node(nfeat, aggs[0], aggs[1], nW1[i, :_D],
                                 nW1[i, _D:], nW2[i], nb1[i], nb2[i],
                                 nln_g[i], nln_b[i],
                                 eW1[j, _D:2 * _D], eW1[j, 2 * _D:])
    return (jnp.concatenate(efs, axis=0), nfeat)


# speedup vs baseline: 5.2795x; 1.0216x over previous
import functools

import jax
import jax.numpy as jnp
from jax import lax
from jax.experimental import pallas as pl
from jax.experimental.pallas import tpu as pltpu
from jax.experimental.pallas import tpu_sc as plsc

_N = 10000
_E = 320000
_D = 128
_L = 4

_NC = 2
_NS = 16
_NW = _NC * _NS
_PER_W = _E // _NW
_B = 80
_NB = _PER_W // _B
_NPAD = 10240
_ZROWS = 128
_AGG_T = _NPAD // _NS
_BLK_N = 1000
_EDGE_GRID = 40


def _sc_gather(ps, pd, src, dst, off, cnt):
    per_w = cnt // _NW
    nb = per_w // _B
    mesh = plsc.VectorSubcoreMesh(core_axis_name="c", subcore_axis_name="s")
    S = 3

    @functools.partial(
        pl.kernel,
        out_type=jax.ShapeDtypeStruct((cnt, _D), jnp.float32),
        mesh=mesh,
        scratch_types=(
            [pltpu.VMEM((per_w,), jnp.int32)] * 2
            + [pltpu.VMEM((_B, _D), jnp.float32)] * (3 * S)
            + [pltpu.SemaphoreType.DMA] * (2 * S)
        ),
    )
    def k(ps_h, pd_h, src_h, dst_h, g_h, idx_s, idx_d, *bufs):
        rows_s = bufs[0:S]
        rows_d = bufs[S:2 * S]
        out = bufs[2 * S:3 * S]
        sem_g = bufs[3 * S:4 * S]
        sem_st = bufs[4 * S:5 * S]
        c = lax.axis_index("c")
        s = lax.axis_index("s")
        base0 = (c * _NS + s) * per_w
        pltpu.sync_copy(src_h.at[pl.ds(off + base0, per_w)], idx_s)
        pltpu.sync_copy(dst_h.at[pl.ds(off + base0, per_w)], idx_d)

        def issue(b, j):
            isl = pl.ds(b * _B, _B)
            pltpu.async_copy(ps_h.at[idx_s.at[isl]], rows_s[j], sem_g[j])
            pltpu.async_copy(pd_h.at[idx_d.at[isl]], rows_d[j], sem_g[j])

        def drain_gather(j):
            pltpu.make_async_copy(ps_h.at[pl.ds(0, _B)], rows_s[j], sem_g[j]).wait()
            pltpu.make_async_copy(pd_h.at[pl.ds(0, _B)], rows_d[j], sem_g[j]).wait()

        def drain_store(j):
            pltpu.make_async_copy(out[j], g_h.at[pl.ds(0, _B)], sem_st[j]).wait()

        def step(b, j, wait_st, look):
            drain_gather(j)
            if wait_st:
                drain_store(j)

            def add_row(r, carry):
                for t in range(_D // 16):
                    sl = pl.ds(t * 16, 16)
                    out[j][r, sl] = rows_s[j][r, sl] + rows_d[j][r, sl]
                return carry

            lax.fori_loop(0, _B, add_row, 0)
            pltpu.async_copy(out[j], g_h.at[pl.ds(base0 + b * _B, _B)], sem_st[j])
            if look:
                issue(b + S - 1, (j + S - 1) % S)

        for j in range(S - 1):
            issue(j, j)
        for j in range(S):
            step(j, j, wait_st=False, look=True)

        def body(ii, carry):
            b = ii * S
            for j in range(S):
                step(b + j, j, wait_st=True, look=True)
            return carry

        n_mid = nb // S - 2
        lax.fori_loop(1, 1 + n_mid, body, 0)
        tail0 = (nb // S - 1) * S
        for b in range(tail0, nb):
            step(b, b % S, wait_st=True, look=(b + S - 1) < nb)
        for j in range(S):
            drain_store(j)

    return k(ps, pd, src, dst)


def _sc_scatter(efeat, dst, off, cnt):
    per_w = cnt // _NW
    nb = per_w // _B
    mesh = plsc.VectorSubcoreMesh(core_axis_name="c", subcore_axis_name="s")
    S = 4

    @functools.partial(
        pl.kernel,
        out_type=jax.ShapeDtypeStruct((_NC, _NPAD, _D), jnp.float32),
        mesh=mesh,
        scratch_types=(
            [pltpu.VMEM((_B,), jnp.int32)] * S
            + [pltpu.VMEM((_B, _D), jnp.float32)] * S
            + [pltpu.VMEM_SHARED((_NPAD, _D), jnp.float32)]
            + [pltpu.SemaphoreType.DMA] * (2 * S)
        ),
    )
    def k(ef_h, dst_h, agg_h, *bufs):
        idx = bufs[0:S]
        rows = bufs[S:2 * S]
        agg_sh = bufs[2 * S]
        sem_l = bufs[2 * S + 1:3 * S + 1]
        sem_sc = bufs[3 * S + 1:4 * S + 1]
        c = lax.axis_index("c")
        s = lax.axis_index("s")
        base0 = (c * _NS + s) * per_w

        def zrow(r, carry):
            for t in range(_D // 16):
                rows[0][r, pl.ds(t * 16, 16)] = jnp.zeros((16,), jnp.float32)
            return carry

        lax.fori_loop(0, _B, zrow, 0)
        for ch in range(_AGG_T // _B):
            pltpu.sync_copy(rows[0], agg_sh.at[pl.ds(s * _AGG_T + ch * _B, _B)])
        plsc.subcore_barrier()

        def load(b, j):
            base = base0 + b * _B
            pltpu.async_copy(dst_h.at[pl.ds(off + base, _B)], idx[j], sem_l[j])
            pltpu.async_copy(ef_h.at[pl.ds(base, _B)], rows[j], sem_l[j])

        def drain_load(j):
            pltpu.make_async_copy(dst_h.at[pl.ds(0, _B)], idx[j], sem_l[j]).wait()
            pltpu.make_async_copy(ef_h.at[pl.ds(0, _B)], rows[j], sem_l[j]).wait()

        def drain_scatter(j):
            pltpu.make_async_copy(ef_h.at[pl.ds(0, _B)], rows[j], sem_sc[j]).wait()

        def step(b, j, wait_sc, look):
            drain_load(j)
            pltpu.async_copy(rows[j], agg_sh.at[idx[j]], sem_sc[j], add=True)
            if wait_sc:
                drain_scatter((j + S - 2) % S)
            if look:
                load(b + S - 2, (j + S - 2) % S)

        for j in range(S - 2):
            load(j, j)
        for j in range(S):
            step(j, j, wait_sc=j >= 2, look=True)

        def body(ii, carry):
            b = ii * S
            for j in range(S):
                step(b + j, j, wait_sc=True, look=True)
            return carry

        lax.fori_loop(1, nb // S - 1, body, 0)
        for b in range((nb // S - 1) * S, nb):
            step(b, b % S, wait_sc=True, look=(b + S - 2) < nb)
        drain_scatter((nb - 1) % S)
        drain_scatter((nb - 2) % S)
        plsc.subcore_barrier()

        for ch in range(_AGG_T // _B):
            o = s * _AGG_T + ch * _B
            pltpu.sync_copy(agg_sh.at[pl.ds(o, _B)], rows[0])
            pltpu.sync_copy(rows[0], agg_h.at[c, pl.ds(o, _B)])

    return k(efeat, dst)


_BLK_E = 2560


def _tc_proj(nfeat, ws, wd):
    blk = _BLK_N

    def body(nf_r, ws_r, wd_r, ps_r, pd_r):
        nf = nf_r[...]
        ps_r[...] = jnp.dot(nf, ws_r[...], preferred_element_type=jnp.float32)
        pd_r[...] = jnp.dot(nf, wd_r[...], preferred_element_type=jnp.float32)

    return pl.pallas_call(
        body,
        grid=(_N // blk,),
        in_specs=[
            pl.BlockSpec((blk, _D), lambda i: (i, 0)),
            pl.BlockSpec((_D, _D), lambda i: (0, 0)),
            pl.BlockSpec((_D, _D), lambda i: (0, 0)),
        ],
        out_specs=[
            pl.BlockSpec((blk, _D), lambda i: (i, 0)),
            pl.BlockSpec((blk, _D), lambda i: (i, 0)),
        ],
        out_shape=[
            jax.ShapeDtypeStruct((_N, _D), jnp.float32),
            jax.ShapeDtypeStruct((_N, _D), jnp.float32),
        ],
    )(nfeat, ws, wd)


def _ln_res(base, h, g, b):
    mu = jnp.mean(h, axis=-1, keepdims=True)
    var = jnp.mean((h - mu) ** 2, axis=-1, keepdims=True)
    return base + ((h - mu) * lax.rsqrt(var + 1e-5)) * g + b


def _tc_edge(efeat, g, we, w2, b1, b2, lng, lnb, off, cnt):
    blk = _BLK_E
    ob = off // blk

    def body(g_r, ef_r, we_r, w2_r, b1_r, b2_r, lng_r, lnb_r, out_r):
        ef = ef_r[...]
        x = jnp.dot(ef, we_r[...], preferred_element_type=jnp.float32)
        x = x + g_r[...] + b1_r[...]
        x = x * jax.nn.sigmoid(x)
        h = jnp.dot(x, w2_r[...], preferred_element_type=jnp.float32) + b2_r[...]
        out_r[...] = _ln_res(ef, h, lng_r[...], lnb_r[...])

    vec = lambda a: a.reshape(1, _D)
    return pl.pallas_call(
        body,
        grid=(cnt // blk,),
        in_specs=[
            pl.BlockSpec((blk, _D), lambda i: (i, 0)),
            pl.BlockSpec((blk, _D), lambda i: (i + ob, 0)),
            pl.BlockSpec((_D, _D), lambda i: (0, 0)),
            pl.BlockSpec((_D, _D), lambda i: (0, 0)),
            pl.BlockSpec((1, _D), lambda i: (0, 0)),
            pl.BlockSpec((1, _D), lambda i: (0, 0)),
            pl.BlockSpec((1, _D), lambda i: (0, 0)),
            pl.BlockSpec((1, _D), lambda i: (0, 0)),
        ],
        out_specs=pl.BlockSpec((blk, _D), lambda i: (i, 0)),
        out_shape=jax.ShapeDtypeStruct((cnt, _D), jnp.float32),
    )(g, efeat, we, w2, vec(b1), vec(b2), vec(lng), vec(lnb))


def _tc_node(nfeat, aggs0, aggs1, w1a, w1b, w2, b1, b2, lng, lnb,
             ws_next, wd_next):
    blk = _BLK_N

    def body(nf_r, a0_r, a1_r, a2_r, a3_r, w1a_r, w1b_r, w2_r, b1_r, b2_r,
             lng_r, lnb_r, wsn_r, wdn_r, nf_o, ps_o, pd_o):
        nf = nf_r[...]
        agg = (a0_r[0] + a1_r[0]) + (a2_r[0] + a3_r[0])
        x = jnp.dot(nf, w1a_r[...], preferred_element_type=jnp.float32)
        x = x + jnp.dot(agg, w1b_r[...], preferred_element_type=jnp.float32)
        x = x + b1_r[...]
        x = x * jax.nn.sigmoid(x)
        h = jnp.dot(x, w2_r[...], preferred_element_type=jnp.float32) + b2_r[...]
        nf_new = _ln_res(nf, h, lng_r[...], lnb_r[...])
        nf_o[...] = nf_new
        ps_o[...] = jnp.dot(nf_new, wsn_r[...], preferred_element_type=jnp.float32)
        pd_o[...] = jnp.dot(nf_new, wdn_r[...], preferred_element_type=jnp.float32)

    vec = lambda a: a.reshape(1, _D)
    row = pl.BlockSpec((blk, _D), lambda i: (i, 0))
    wsp = pl.BlockSpec((_D, _D), lambda i: (0, 0))
    bsp = pl.BlockSpec((1, _D), lambda i: (0, 0))
    return pl.pallas_call(
        body,
        grid=(_N // blk,),
        in_specs=[
            row,
            pl.BlockSpec((1, blk, _D), lambda i: (0, i, 0)),
            pl.BlockSpec((1, blk, _D), lambda i: (1, i, 0)),
            pl.BlockSpec((1, blk, _D), lambda i: (0, i, 0)),
            pl.BlockSpec((1, blk, _D), lambda i: (1, i, 0)),
            wsp, wsp, wsp, bsp, bsp, bsp, bsp, wsp, wsp,
        ],
        out_specs=[row, row, row],
        out_shape=[
            jax.ShapeDtypeStruct((_N, _D), jnp.float32),
            jax.ShapeDtypeStruct((_N, _D), jnp.float32),
            jax.ShapeDtypeStruct((_N, _D), jnp.float32),
        ],
    )(nfeat, aggs0, aggs0, aggs1, aggs1, w1a, w1b, w2, vec(b1), vec(b2),
      vec(lng), vec(lnb), ws_next, wd_next)


_C0 = 158720


def kernel(efeat, nfeat, edge_index, eW1, eb1, eW2, eb2, eln_g, eln_b,
           nW1, nb1, nW2, nb2, nln_g, nln_b):
    src = edge_index[0]
    dst = edge_index[1]
    chunks = ((0, _C0), (_C0, _E - _C0))
    efs = [None, None]
    ps, pd = _tc_proj(nfeat, eW1[0, _D:2 * _D], eW1[0, 2 * _D:])
    for i in range(_L):
        gs = [None, None]
        aggs = [None, None]
        for k, (off, cnt) in enumerate(chunks):
            gs[k] = _sc_gather(ps, pd, src, dst, off, cnt)
        for k, (off, cnt) in enumerate(chunks):
            src_arr = efeat if i == 0 else efs[k]
            src_off = off if i == 0 else 0
            efs[k] = _tc_edge(src_arr, gs[k], eW1[i, :_D], eW2[i], eb1[i],
                              eb2[i], eln_g[i], eln_b[i], src_off, cnt)
            aggs[k] = _sc_scatter(efs[k], dst, off, cnt)
        j = (i + 1) % _L
        nfeat, ps, pd = _tc_node(nfeat, aggs[0], aggs[1], nW1[i, :_D],
                                 nW1[i, _D:], nW2[i], nb1[i], nb2[i],
                                 nln_g[i], nln_b[i],
                                 eW1[j, _D:2 * _D], eW1[j, 2 * _D:])
    return (jnp.concatenate(efs, axis=0), nfeat)
